# Initial kernel scaffold; baseline (speedup 1.0000x reference)
import jax
import jax.numpy as jnp
from jax.experimental import pallas as pl


def kernel(elec_elec_dists, elec_nuc_dists, nuc_nuc_dists, flat_charges, W1_same, b1_same, W2_same, W1_diff, b1_diff, W2_diff, W_ee, b_ee, kernel, bias_nuc, W1_en, b1_en, W2_en, W_scale, Ws, bs, Wd, bd, bias_u, W_o, b_o, W_w, W_f, b_f, elec_elec_idx, elec_nuc_idx_elec, elec_nuc_idx_nuc, nuc_nuc_idx, spin_mask):
    raise NotImplementedError("write your pallas kernel here")


# speedup vs baseline: 1.0395x; 1.0395x over previous
import functools

import jax
import jax.numpy as jnp
import numpy as np
from jax import lax
from jax.experimental import pallas as pl
from jax.experimental.pallas import tpu as pltpu

N_ELEC = 10000
N_NUC = 1000
N_EE = 160000
N_SAME = 80000
N_EN = 160000
N_NN = 16000
EMB = 128
DIM = 128
EDGE_EMB = 16
HID = 32
RBF = 16
N_LAYER = 3

_EDGE_BLK = 2000
_SQRT2 = 1.4142135623730951


def _silu(x):
    return x * jax.nn.sigmoid(x)


def _stage_a_body(ee_d_ref, en_d_ref,
                  W1s_ref, b1s_ref, W2s_ref, W1d_ref, b1d_ref, W2d_ref,
                  Wee_ref, bee_ref, W1en_ref, b1en_ref, W2en_ref,
                  Wsc_ref, Ww_ref,
                  vee_ref, gen_ref, eenv_ref, scale_ref, wedge_ref):
    pid = pl.program_id(0)
    sigma2 = (lax.broadcasted_iota(jnp.int32, (1, RBF), 1).astype(jnp.float32)
              * jnp.float32(4.5 / (RBF - 1)) + 0.5)

    ee_d = ee_d_ref[...]
    r_ee = ee_d[:, 3]
    feats = jnp.exp(-r_ee[:, None] * sigma2)
    is_same = pid * _EDGE_BLK < N_SAME
    W1 = jnp.where(is_same, W1s_ref[...], W1d_ref[...])
    b1 = jnp.where(is_same, b1s_ref[...], b1d_ref[...])
    W2 = jnp.where(is_same, W2s_ref[...], W2d_ref[...])
    h = _silu(jnp.dot(feats, W1, preferred_element_type=jnp.float32)
              + b1[None, :])
    filt = jnp.dot(h, W2, preferred_element_type=jnp.float32)
    g_ee = jnp.log1p(r_ee) / (r_ee + 1e-12)
    data = _silu(jnp.dot(ee_d * g_ee[:, None], Wee_ref[...],
                         preferred_element_type=jnp.float32) + bee_ref[...][None, :])
    vee_ref[...] = filt * data

    en_d = en_d_ref[...]
    r_en = en_d[:, 3]
    gen_ref[0, 0, :] = jnp.log1p(r_en) / (r_en + 1e-12)
    eenv_ref[0, 0, :] = jnp.exp(-r_en)
    feats_en = jnp.exp(-r_en[:, None] * sigma2)
    h_en = _silu(jnp.dot(feats_en, W1en_ref[...],
                         preferred_element_type=jnp.float32) + b1en_ref[...][None, :])
    edge16 = jnp.dot(h_en, W2en_ref[...], preferred_element_type=jnp.float32)
    scale_ref[...] = jnp.dot(edge16, Wsc_ref[...],
                             preferred_element_type=jnp.float32)
    wedge_ref[...] = jnp.dot(edge16, Ww_ref[...],
                             preferred_element_type=jnp.float32)


def _stage_a(ee_dists, en_dists, W1s, b1s, W2s, W1d, b1d, W2d, Wee, bee,
             W1en, b1en, W2en, Wsc, Ww):
    n_blk = N_EE // _EDGE_BLK
    blk = _EDGE_BLK
    full = lambda shape: pl.BlockSpec(shape, lambda i: tuple(0 for _ in shape))
    return pl.pallas_call(
        _stage_a_body,
        grid=(n_blk,),
        in_specs=[
            pl.BlockSpec((blk, 4), lambda i: (i, 0)),
            pl.BlockSpec((blk, 4), lambda i: (i, 0)),
            full((RBF, HID)), full((HID,)), full((HID, EMB // 2)),
            full((RBF, HID)), full((HID,)), full((HID, EMB // 2)),
            full((4, EMB // 2)), full((EMB // 2,)),
            full((RBF, HID)), full((HID,)), full((HID, EDGE_EMB)),
            full((EDGE_EMB, 2 * EMB)), full((EDGE_EMB, DIM)),
        ],
        out_specs=[
            pl.BlockSpec((blk, EMB // 2), lambda i: (i, 0)),
            pl.BlockSpec((1, 1, blk), lambda i: (i, 0, 0)),
            pl.BlockSpec((1, 1, blk), lambda i: (i, 0, 0)),
            pl.BlockSpec((blk, 2 * EMB), lambda i: (i, 0)),
            pl.BlockSpec((blk, DIM), lambda i: (i, 0)),
        ],
        out_shape=[
            jax.ShapeDtypeStruct((N_EE, EMB // 2), jnp.float32),
            jax.ShapeDtypeStruct((N_EN // _EDGE_BLK, 1, _EDGE_BLK), jnp.float32),
            jax.ShapeDtypeStruct((N_EN // _EDGE_BLK, 1, _EDGE_BLK), jnp.float32),
            jax.ShapeDtypeStruct((N_EN, 2 * EMB), jnp.float32),
            jax.ShapeDtypeStruct((N_EN, DIM), jnp.float32),
        ],
    )(ee_dists, en_dists, W1s, b1s, W2s, W1d, b1d, W2d, Wee, bee,
      W1en, b1en, W2en, Wsc, Ww)


def _stage_a2_body(nn_d_ref, out_ref):
    out_ref[...] = jnp.exp(-nn_d_ref[:, 3])


def _stage_a2(nn_dists):
    return pl.pallas_call(
        _stage_a2_body,
        out_shape=jax.ShapeDtypeStruct((N_NN,), jnp.float32),
    )(nn_dists)


def _stage_p_body(ee0_ref, ee1_ref, nc_ref, nn_ref, es_ref, invn_ref,
                  invnn_ref):
    lo = ee0_ref[...]
    hi = ee1_ref[...]
    emb = jnp.concatenate([lo[0] + lo[1], hi[0] + hi[1]], axis=1)
    norm = nc_ref[0, :N_ELEC] + nc_ref[1, :N_ELEC] + 1.0
    inv = 1.0 / norm
    invn_ref[...] = inv
    es_ref[...] = emb * inv[:, None]
    nn = nn_ref[0, :N_NUC] + nn_ref[1, :N_NUC] + 1.0
    invnn_ref[...] = 1.0 / nn


def _stage_p(ee_acc, norm_acc, nn_acc):
    return pl.pallas_call(
        _stage_p_body,
        out_shape=[
            jax.ShapeDtypeStruct((N_ELEC, EMB), jnp.float32),
            jax.ShapeDtypeStruct((N_ELEC,), jnp.float32),
            jax.ShapeDtypeStruct((N_NUC,), jnp.float32),
        ],
    )(ee_acc[:, :N_ELEC], ee_acc[:, N_ELEC:], norm_acc, nn_acc)


def _stage_e_body(aggE0_ref, aggE1_ref, aggN_ref, invn_ref, invnn_ref,
                  Ws_ref, bs_ref, Wd_ref, bd_ref, bu_ref, Wo_ref, bo_ref,
                  elec_ref, outpre_ref, ud_ref):
    inv = invn_ref[...]
    elec = (aggE0_ref[...] + aggE1_ref[...]) * inv[:, None]
    elec_ref[...] = elec
    outpre_ref[...] = (jnp.dot(elec, Wo_ref[...],
                               preferred_element_type=jnp.float32)
                       + bo_ref[...][None, :])
    aggN = aggN_ref[...]
    nuc = (aggN[0] + aggN[1]) * invnn_ref[...].reshape(1, N_NUC, 1)
    up, down = nuc[0], nuc[1]
    for l in range(N_LAYER):
        su = jnp.dot(up, Ws_ref[l], preferred_element_type=jnp.float32)
        du = jnp.dot(up, Wd_ref[l], preferred_element_type=jnp.float32)
        sd = jnp.dot(down, Ws_ref[l], preferred_element_type=jnp.float32)
        dd = jnp.dot(down, Wd_ref[l], preferred_element_type=jnp.float32)
        bias = bs_ref[l][None, :] + bd_ref[l][None, :]
        pre_u = (su + dd + bias) / _SQRT2 + bu_ref[l]
        pre_d = (sd + du + bias) / _SQRT2 + bu_ref[l]
        up = (up + _silu(pre_u)) / _SQRT2
        down = (down + _silu(pre_d)) / _SQRT2
    ud_ref[...] = jnp.concatenate([up, down], axis=0)


def _stage_e(aggE, aggN, inv_norm, inv_nneigh, Ws, bs, Wd, bd, bias_u,
             W_o, b_o):
    return pl.pallas_call(
        _stage_e_body,
        out_shape=[
            jax.ShapeDtypeStruct((N_ELEC, EMB), jnp.float32),
            jax.ShapeDtypeStruct((N_ELEC, DIM), jnp.float32),
            jax.ShapeDtypeStruct((2 * N_NUC, DIM), jnp.float32),
        ],
    )(aggE[0], aggE[1], aggN, inv_norm, inv_nneigh, Ws, bs, Wd, bd,
      bias_u, W_o, b_o)


def _stage_g_body(d0_ref, d1_ref, invn_ref, elec_ref, outpre_ref,
                  Wf_ref, bf_ref, out_ref):
    diff = (d0_ref[...] + d1_ref[...]) * invn_ref[...][:, None]
    o = _silu(outpre_ref[...] * diff)
    o = _silu(jnp.dot(o, Wf_ref[...], preferred_element_type=jnp.float32)
              + bf_ref[...][None, :])
    out_ref[...] = (elec_ref[...] + o) / _SQRT2


def _stage_g(aggD, inv_norm, elec_emb, out_pre, W_f, b_f):
    return pl.pallas_call(
        _stage_g_body,
        out_shape=jax.ShapeDtypeStruct((N_ELEC, EMB), jnp.float32),
    )(aggD[0], aggD[1], inv_norm, elec_emb, out_pre, W_f, b_f)


def _seg_sum(vals, idx, num):
    return jax.ops.segment_sum(vals, idx, num_segments=num)


def kernel(elec_elec_dists, elec_nuc_dists, nuc_nuc_dists, flat_charges,
           W1_same, b1_same, W2_same, W1_diff, b1_diff, W2_diff, W_ee, b_ee,
           kernel, bias_nuc, W1_en, b1_en, W2_en, W_scale,
           Ws, bs, Wd, bd, bias_u, W_o, b_o, W_w, W_f, b_f,
           elec_elec_idx, elec_nuc_idx_elec, elec_nuc_idx_nuc, nuc_nuc_idx,
           spin_mask):
    f32 = jnp.float32
    v_ee, g_en, e_env, scale, w_edge = _stage_a(
        elec_elec_dists, elec_nuc_dists, W1_same, b1_same, W2_same,
        W1_diff, b1_diff, W2_diff, W_ee, b_ee, W1_en, b1_en, W2_en,
        W_scale, W_w)
    g_en = g_en.reshape(N_EN)
    e_env = e_env.reshape(N_EN)
    nn_env = _stage_a2(nuc_nuc_dists)

    i32 = jnp.int32
    e_idx = elec_nuc_idx_elec.astype(i32)
    n_idx = elec_nuc_idx_nuc.astype(i32)
    ee_i = elec_elec_idx[0].astype(i32)
    idx_ee = jnp.concatenate([ee_i[:N_SAME], ee_i[N_SAME:] + N_ELEC])
    mask = spin_mask[e_idx]
    nidx2 = n_idx + N_NUC * mask.astype(i32)
    ud_idx = n_idx + N_NUC * (1 - mask.astype(i32))

    e_emb2 = _seg_sum(v_ee, idx_ee, 2 * N_ELEC)
    ee_acc = jnp.stack([e_emb2, jnp.zeros_like(e_emb2)])
    normc = _seg_sum(e_env * flat_charges[n_idx], e_idx, N_ELEC)
    norm_acc = jnp.stack([normc, jnp.zeros_like(normc)])
    nnc = _seg_sum(nn_env * flat_charges[nuc_nuc_idx[1]],
                   nuc_nuc_idx[0].astype(i32), N_NUC)
    nn_acc = jnp.stack([nnc, jnp.zeros_like(nnc)])

    ES, inv_norm, inv_nneigh = _stage_p(ee_acc, norm_acc, nn_acc)

    KB = jnp.concatenate([kernel.reshape(N_NUC, 4 * EMB), bias_nuc], axis=1)
    kb_rows = KB[n_idx]
    contract = jnp.einsum('ed,edk->ek', elec_nuc_dists,
                          kb_rows[:, :512].reshape(N_EN, 4, EMB))
    en_emb = _silu(g_en[:, None] * contract + kb_rows[:, 512:] + ES[e_idx])
    p0 = en_emb * scale[:, :EMB]
    p1 = en_emb * scale[:, EMB:]
    aggE = jnp.stack([_seg_sum(p0, e_idx, N_ELEC),
                      jnp.zeros((N_ELEC, EMB), f32)])
    aggN = _seg_sum(p1, nidx2, 2 * N_NUC)
    aggN = jnp.stack([aggN, jnp.zeros_like(aggN)]).reshape(2, 2, N_NUC, EMB)

    elec_emb, out_pre, UD = _stage_e(aggE, aggN, inv_norm, inv_nneigh,
                                     Ws, bs, Wd, bd, bias_u, W_o, b_o)

    to_elec = UD[ud_idx] * w_edge
    aggD = jnp.stack([_seg_sum(to_elec, e_idx, N_ELEC),
                      jnp.zeros((N_ELEC, EMB), f32)])

    return _stage_g(aggD, inv_norm, elec_emb, out_pre, W_f, b_f)
